# baseline (speedup 1.0000x reference)
import functools

import jax
import jax.numpy as jnp
from jax import lax
from jax.experimental import pallas as pl
from jax.experimental.pallas import tpu as pltpu
from jax.experimental.pallas import tpu_sc as plsc

R, C = 1024, 100000
SUB = 8
LANE = 128
_NEG_INF = float("-inf")
_BIG = 1 << 30

C_SC = 41600
C_TC = C - C_SC

RT = 2336
NJ = C_TC // RT

NCORES, NSUB = 2, 16
NW = NCORES * NSUB
NR4 = 4
NLB = 8
RPW = C_SC // NR4
RTS = 200
NCH = RPW // RTS
PAIRS = NCH // 2


def _tc_body(x_ref, ov_ref, oi_ref, m_ref, c_ref):
    j = pl.program_id(0)

    @pl.when(j == 0)
    def _():
        m_ref[...] = jnp.full((SUB, R), _NEG_INF, dtype=jnp.float32)
        c_ref[...] = jnp.zeros((SUB, R), dtype=jnp.int32)

    m = m_ref[...]
    c = c_ref[...]
    for k in range(RT // SUB):
        v = x_ref[pl.ds(SUB * k, SUB), :]
        p = v > m
        m = jnp.where(p, v, m)
        c = jnp.where(p, j * RT + SUB * k, c)
    m_ref[...] = m
    c_ref[...] = c

    @pl.when(j == NJ - 1)
    def _():
        mm = jnp.max(m, axis=0, keepdims=True)
        srow = lax.broadcasted_iota(jnp.int32, (SUB, R), 0)
        cand = jnp.where(m == mm, c + srow, _BIG)
        ov_ref[...] = mm
        oi_ref[...] = jnp.min(cand, axis=0, keepdims=True)


def _argmax_tc(xt):
    return pl.pallas_call(
        _tc_body,
        grid=(NJ,),
        in_specs=[pl.BlockSpec((RT, R), lambda j: (j, 0))],
        out_specs=[
            pl.BlockSpec((1, R), lambda j: (0, 0)),
            pl.BlockSpec((1, R), lambda j: (0, 0)),
        ],
        out_shape=[
            jax.ShapeDtypeStruct((1, R), jnp.float32),
            jax.ShapeDtypeStruct((1, R), jnp.int32),
        ],
        scratch_shapes=[
            pltpu.VMEM((SUB, R), jnp.float32),
            pltpu.VMEM((SUB, R), jnp.int32),
        ],
        compiler_params=pltpu.CompilerParams(
            dimension_semantics=("arbitrary",),
        ),
    )(xt)


def _partial_sc(xt):
    mesh = plsc.VectorSubcoreMesh(core_axis_name="c", subcore_axis_name="s")

    @functools.partial(
        pl.kernel,
        out_type=(
            jax.ShapeDtypeStruct((NR4, NLB, LANE), jnp.float32),
            jax.ShapeDtypeStruct((NR4, NLB, LANE), jnp.int32),
        ),
        mesh=mesh,
        compiler_params=pltpu.CompilerParams(
            needs_layout_passes=False, use_tc_tiling_on_sc=True
        ),
        scratch_types=[
            pltpu.VMEM((2, RTS, LANE), jnp.float32),
            pltpu.VMEM((LANE,), jnp.float32),
            pltpu.VMEM((LANE,), jnp.int32),
            pltpu.SemaphoreType.DMA,
            pltpu.SemaphoreType.DMA,
        ],
    )
    def k(xt_hbm, val_hbm, idx_hbm, buf, vstage, istage, sem0, sem1):
        cid = lax.axis_index("c")
        sid = lax.axis_index("s")
        wid = sid * NCORES + cid
        lb = wid % NLB
        r4 = wid // NLB
        row0 = C_TC + r4 * RPW

        def chunk_copy(ci, slot):
            sem = sem0 if slot == 0 else sem1
            src = xt_hbm.at[pl.ds(row0 + ci * RTS, RTS), pl.ds(lb * LANE, LANE)]
            return pltpu.make_async_copy(src, buf.at[slot], sem)

        chunk_copy(0, 0).start()
        chunk_copy(1, 1).start()

        neg = jnp.full((16,), _NEG_INF, dtype=jnp.float32)
        zero = jnp.zeros((16,), dtype=jnp.int32)

        def compute_chunk(slot, colbase, st):
            def body(rt, st_):
                out = list(st_)
                for s in range(SUB):
                    col = colbase + rt * SUB + s
                    for kk in range(8):
                        rm, rc = out[2 * kk], out[2 * kk + 1]
                        v = buf[slot, rt * SUB + s, pl.ds(kk * 16, 16)]
                        p = v > rm
                        out[2 * kk] = jnp.where(p, v, rm)
                        out[2 * kk + 1] = jnp.where(p, col, rc)
                return tuple(out)

            return lax.fori_loop(0, RTS // SUB, body, tuple(st))

        def pair_body(t, st):
            c0 = 2 * t
            chunk_copy(c0, 0).wait()
            st = compute_chunk(0, row0 + c0 * RTS, st)

            @pl.when(t < PAIRS - 1)
            def _():
                chunk_copy(c0 + 2, 0).start()

            chunk_copy(c0 + 1, 1).wait()
            st = compute_chunk(1, row0 + (c0 + 1) * RTS, st)

            @pl.when(t < PAIRS - 1)
            def _():
                chunk_copy(c0 + 3, 1).start()

            return st

        st = lax.fori_loop(0, PAIRS, pair_body, (neg, zero) * 8)
        for kk in range(8):
            vstage[pl.ds(kk * 16, 16)] = st[2 * kk]
            istage[pl.ds(kk * 16, 16)] = st[2 * kk + 1]
        pltpu.sync_copy(vstage, val_hbm.at[r4, lb])
        pltpu.sync_copy(istage, idx_hbm.at[r4, lb])

    return k(xt)


def _merge_body(tv_ref, ti_ref, sv_ref, si_ref, o_ref):
    bv = tv_ref[...]
    bi = ti_ref[...]
    for r in range(NR4):
        sv = sv_ref[pl.ds(r, 1), :]
        si = si_ref[pl.ds(r, 1), :]
        p = sv > bv
        bv = jnp.where(p, sv, bv)
        bi = jnp.where(p, si, bi)
    o_ref[...] = bi


def _merge(tc_val, tc_idx, sc_val, sc_idx):
    return pl.pallas_call(
        _merge_body,
        out_shape=jax.ShapeDtypeStruct((1, R), jnp.int32),
    )(tc_val, tc_idx, sc_val, sc_idx)


def kernel(inputs):
    xt = jnp.swapaxes(inputs, 0, 1)
    sc_val, sc_idx = _partial_sc(xt)
    tc_val, tc_idx = _argmax_tc(xt)
    out = _merge(
        tc_val, tc_idx, sc_val.reshape(NR4, R), sc_idx.reshape(NR4, R)
    )
    return out.reshape(R)


# speedup vs baseline: 1.0235x; 1.0235x over previous
import functools

import jax
import jax.numpy as jnp
from jax import lax
from jax.experimental import pallas as pl
from jax.experimental.pallas import tpu as pltpu
from jax.experimental.pallas import tpu_sc as plsc

R, C = 1024, 100000
SUB = 8
LANE = 128
_NEG_INF = float("-inf")
_BIG = 1 << 30

C_SC = 40000
C_TC = C - C_SC

RT = 2000
NJ = C_TC // RT

NCORES, NSUB = 2, 16
NW = NCORES * NSUB
NR4 = 4
NLB = 8
RPW = C_SC // NR4
RTS = 200
NCH = RPW // RTS
PAIRS = NCH // 2


NJ2 = NJ // 2


def _tc_body(x1_ref, x2_ref, ov_ref, oi_ref, m_ref, c_ref):
    j = pl.program_id(0)

    @pl.when(j == 0)
    def _():
        m_ref[...] = jnp.full((2, SUB, R), _NEG_INF, dtype=jnp.float32)
        c_ref[...] = jnp.zeros((2, SUB, R), dtype=jnp.int32)

    for h, x_ref in ((0, x1_ref), (1, x2_ref)):
        m = m_ref[h]
        c = c_ref[h]
        base = (j + h * NJ2) * RT
        for k in range(RT // SUB):
            v = x_ref[pl.ds(SUB * k, SUB), :]
            p = v > m
            m = jnp.where(p, v, m)
            c = jnp.where(p, base + SUB * k, c)
        m_ref[h] = m
        c_ref[h] = c

    @pl.when(j == NJ2 - 1)
    def _():
        ma, ca = m_ref[0], c_ref[0]
        mb, cb = m_ref[1], c_ref[1]
        p = mb > ma
        m = jnp.where(p, mb, ma)
        c = jnp.where(p, cb, ca)
        mm = jnp.max(m, axis=0, keepdims=True)
        srow = lax.broadcasted_iota(jnp.int32, (SUB, R), 0)
        cand = jnp.where(m == mm, c + srow, _BIG)
        ov_ref[...] = mm
        oi_ref[...] = jnp.min(cand, axis=0, keepdims=True)


def _argmax_tc(xt):
    return pl.pallas_call(
        _tc_body,
        grid=(NJ2,),
        in_specs=[
            pl.BlockSpec((RT, R), lambda j: (j, 0)),
            pl.BlockSpec((RT, R), lambda j: (j + NJ2, 0)),
        ],
        out_specs=[
            pl.BlockSpec((1, R), lambda j: (0, 0)),
            pl.BlockSpec((1, R), lambda j: (0, 0)),
        ],
        out_shape=[
            jax.ShapeDtypeStruct((1, R), jnp.float32),
            jax.ShapeDtypeStruct((1, R), jnp.int32),
        ],
        scratch_shapes=[
            pltpu.VMEM((2, SUB, R), jnp.float32),
            pltpu.VMEM((2, SUB, R), jnp.int32),
        ],
        compiler_params=pltpu.CompilerParams(
            dimension_semantics=("arbitrary",),
        ),
    )(xt, xt)


def _partial_sc(xt):
    mesh = plsc.VectorSubcoreMesh(core_axis_name="c", subcore_axis_name="s")

    @functools.partial(
        pl.kernel,
        out_type=(
            jax.ShapeDtypeStruct((NR4, NLB, LANE), jnp.float32),
            jax.ShapeDtypeStruct((NR4, NLB, LANE), jnp.int32),
        ),
        mesh=mesh,
        compiler_params=pltpu.CompilerParams(
            needs_layout_passes=False, use_tc_tiling_on_sc=True
        ),
        scratch_types=[
            pltpu.VMEM((2, RTS, LANE), jnp.float32),
            pltpu.VMEM((LANE,), jnp.float32),
            pltpu.VMEM((LANE,), jnp.int32),
            pltpu.SemaphoreType.DMA,
            pltpu.SemaphoreType.DMA,
        ],
    )
    def k(xt_hbm, val_hbm, idx_hbm, buf, vstage, istage, sem0, sem1):
        cid = lax.axis_index("c")
        sid = lax.axis_index("s")
        wid = sid * NCORES + cid
        lb = wid % NLB
        r4 = wid // NLB
        row0 = C_TC + r4 * RPW

        def chunk_copy(ci, slot):
            sem = sem0 if slot == 0 else sem1
            src = xt_hbm.at[pl.ds(row0 + ci * RTS, RTS), pl.ds(lb * LANE, LANE)]
            return pltpu.make_async_copy(src, buf.at[slot], sem)

        chunk_copy(0, 0).start()
        chunk_copy(1, 1).start()

        neg = jnp.full((16,), _NEG_INF, dtype=jnp.float32)
        zero = jnp.zeros((16,), dtype=jnp.int32)

        def compute_chunk(slot, colbase, st):
            def body(rt, st_):
                out = list(st_)
                for s in range(SUB):
                    col = colbase + rt * SUB + s
                    for kk in range(8):
                        rm, rc = out[2 * kk], out[2 * kk + 1]
                        v = buf[slot, rt * SUB + s, pl.ds(kk * 16, 16)]
                        p = v > rm
                        out[2 * kk] = jnp.where(p, v, rm)
                        out[2 * kk + 1] = jnp.where(p, col, rc)
                return tuple(out)

            return lax.fori_loop(0, RTS // SUB, body, tuple(st))

        def pair_body(t, st):
            c0 = 2 * t
            chunk_copy(c0, 0).wait()
            st = compute_chunk(0, row0 + c0 * RTS, st)

            @pl.when(t < PAIRS - 1)
            def _():
                chunk_copy(c0 + 2, 0).start()

            chunk_copy(c0 + 1, 1).wait()
            st = compute_chunk(1, row0 + (c0 + 1) * RTS, st)

            @pl.when(t < PAIRS - 1)
            def _():
                chunk_copy(c0 + 3, 1).start()

            return st

        st = lax.fori_loop(0, PAIRS, pair_body, (neg, zero) * 8)
        for kk in range(8):
            vstage[pl.ds(kk * 16, 16)] = st[2 * kk]
            istage[pl.ds(kk * 16, 16)] = st[2 * kk + 1]
        pltpu.sync_copy(vstage, val_hbm.at[r4, lb])
        pltpu.sync_copy(istage, idx_hbm.at[r4, lb])

    return k(xt)


def _merge_body(tv_ref, ti_ref, sv_ref, si_ref, o_ref):
    bv = tv_ref[...]
    bi = ti_ref[...]
    for r in range(NR4):
        sv = sv_ref[pl.ds(r, 1), :]
        si = si_ref[pl.ds(r, 1), :]
        p = sv > bv
        bv = jnp.where(p, sv, bv)
        bi = jnp.where(p, si, bi)
    o_ref[...] = bi


def _merge(tc_val, tc_idx, sc_val, sc_idx):
    return pl.pallas_call(
        _merge_body,
        out_shape=jax.ShapeDtypeStruct((1, R), jnp.int32),
    )(tc_val, tc_idx, sc_val, sc_idx)


def kernel(inputs):
    xt = jnp.swapaxes(inputs, 0, 1)
    sc_val, sc_idx = _partial_sc(xt)
    tc_val, tc_idx = _argmax_tc(xt)
    out = _merge(
        tc_val, tc_idx, sc_val.reshape(NR4, R), sc_idx.reshape(NR4, R)
    )
    return out.reshape(R)


# speedup vs baseline: 1.0439x; 1.0199x over previous
import functools

import jax
import jax.numpy as jnp
from jax import lax
from jax.experimental import pallas as pl
from jax.experimental.pallas import tpu as pltpu
from jax.experimental.pallas import tpu_sc as plsc

R, C = 1024, 100000
SUB = 8
LANE = 128
_NEG_INF = float("-inf")
_BIG = 1 << 30

C_SC = 40000
C_TC = C - C_SC

RT = 2000
NJ = C_TC // RT

NCORES, NSUB = 2, 16
NW = NCORES * NSUB
NR4 = 4
NLB = 8
RPW = C_SC // NR4
RTS = 200
NCH = RPW // RTS
PAIRS = NCH // 2


def _tc_body(x_ref, ov_ref, oi_ref, m_ref, c_ref):
    j = pl.program_id(0)

    @pl.when(j == 0)
    def _():
        m_ref[...] = jnp.full((SUB, R), _NEG_INF, dtype=jnp.float32)
        c_ref[...] = jnp.zeros((SUB, R), dtype=jnp.int32)

    m = m_ref[...]
    c = c_ref[...]
    for k in range(RT // SUB):
        v = x_ref[pl.ds(SUB * k, SUB), :]
        p = v > m
        m = jnp.where(p, v, m)
        c = jnp.where(p, j * RT + SUB * k, c)
    m_ref[...] = m
    c_ref[...] = c

    @pl.when(j == NJ - 1)
    def _():
        mm = jnp.max(m, axis=0, keepdims=True)
        srow = lax.broadcasted_iota(jnp.int32, (SUB, R), 0)
        cand = jnp.where(m == mm, c + srow, _BIG)
        ov_ref[...] = mm
        oi_ref[...] = jnp.min(cand, axis=0, keepdims=True)


def _argmax_tc(xt):
    return pl.pallas_call(
        _tc_body,
        grid=(NJ,),
        in_specs=[pl.BlockSpec((RT, R), lambda j: (j, 0))],
        out_specs=[
            pl.BlockSpec((1, R), lambda j: (0, 0)),
            pl.BlockSpec((1, R), lambda j: (0, 0)),
        ],
        out_shape=[
            jax.ShapeDtypeStruct((1, R), jnp.float32),
            jax.ShapeDtypeStruct((1, R), jnp.int32),
        ],
        scratch_shapes=[
            pltpu.VMEM((SUB, R), jnp.float32),
            pltpu.VMEM((SUB, R), jnp.int32),
        ],
        compiler_params=pltpu.CompilerParams(
            dimension_semantics=("arbitrary",),
        ),
    )(xt)


def _partial_sc(xt):
    mesh = plsc.VectorSubcoreMesh(core_axis_name="c", subcore_axis_name="s")

    @functools.partial(
        pl.kernel,
        out_type=(
            jax.ShapeDtypeStruct((NR4, R), jnp.float32),
            jax.ShapeDtypeStruct((NR4, R), jnp.int32),
        ),
        mesh=mesh,
        compiler_params=pltpu.CompilerParams(
            needs_layout_passes=False, use_tc_tiling_on_sc=True
        ),
        scratch_types=[
            pltpu.VMEM((2, RTS, LANE), jnp.float32),
            pltpu.VMEM((LANE,), jnp.float32),
            pltpu.VMEM((LANE,), jnp.int32),
            pltpu.SemaphoreType.DMA,
            pltpu.SemaphoreType.DMA,
        ],
    )
    def k(xt_hbm, val_hbm, idx_hbm, buf, vstage, istage, sem0, sem1):
        cid = lax.axis_index("c")
        sid = lax.axis_index("s")
        wid = sid * NCORES + cid
        lb = wid % NLB
        r4 = wid // NLB
        row0 = C_TC + r4 * RPW

        def chunk_copy(ci, slot):
            sem = sem0 if slot == 0 else sem1
            src = xt_hbm.at[pl.ds(row0 + ci * RTS, RTS), pl.ds(lb * LANE, LANE)]
            return pltpu.make_async_copy(src, buf.at[slot], sem)

        chunk_copy(0, 0).start()
        chunk_copy(1, 1).start()

        neg = jnp.full((16,), _NEG_INF, dtype=jnp.float32)
        zero = jnp.zeros((16,), dtype=jnp.int32)

        def compute_chunk(slot, colbase, st):
            def body(rt, st_):
                out = list(st_)
                for s in range(SUB):
                    col = colbase + rt * SUB + s
                    for kk in range(8):
                        rm, rc = out[2 * kk], out[2 * kk + 1]
                        v = buf[slot, rt * SUB + s, pl.ds(kk * 16, 16)]
                        p = v > rm
                        out[2 * kk] = jnp.where(p, v, rm)
                        out[2 * kk + 1] = jnp.where(p, col, rc)
                return tuple(out)

            return lax.fori_loop(0, RTS // SUB, body, tuple(st))

        def pair_body(t, st):
            c0 = 2 * t
            chunk_copy(c0, 0).wait()
            st = compute_chunk(0, row0 + c0 * RTS, st)

            @pl.when(t < PAIRS - 1)
            def _():
                chunk_copy(c0 + 2, 0).start()

            chunk_copy(c0 + 1, 1).wait()
            st = compute_chunk(1, row0 + (c0 + 1) * RTS, st)

            @pl.when(t < PAIRS - 1)
            def _():
                chunk_copy(c0 + 3, 1).start()

            return st

        st = lax.fori_loop(0, PAIRS, pair_body, (neg, zero) * 8)
        for kk in range(8):
            vstage[pl.ds(kk * 16, 16)] = st[2 * kk]
            istage[pl.ds(kk * 16, 16)] = st[2 * kk + 1]
        pltpu.sync_copy(vstage, val_hbm.at[r4, pl.ds(lb * LANE, LANE)])
        pltpu.sync_copy(istage, idx_hbm.at[r4, pl.ds(lb * LANE, LANE)])

    return k(xt)


def _merge_body(tv_ref, ti_ref, sv_ref, si_ref, o_ref):
    bv = tv_ref[...]
    bi = ti_ref[...]
    for r in range(NR4):
        sv = sv_ref[pl.ds(r, 1), :]
        si = si_ref[pl.ds(r, 1), :]
        p = sv > bv
        bv = jnp.where(p, sv, bv)
        bi = jnp.where(p, si, bi)
    o_ref[...] = bi


def _merge(tc_val, tc_idx, sc_val, sc_idx):
    return pl.pallas_call(
        _merge_body,
        out_shape=jax.ShapeDtypeStruct((1, R), jnp.int32),
    )(tc_val, tc_idx, sc_val, sc_idx)


def kernel(inputs):
    xt = jnp.swapaxes(inputs, 0, 1)
    sc_val, sc_idx = _partial_sc(xt)
    tc_val, tc_idx = _argmax_tc(xt)
    out = _merge(tc_val, tc_idx, sc_val, sc_idx)
    return out.reshape(R)
